# Initial kernel scaffold; baseline (speedup 1.0000x reference)
import jax
import jax.numpy as jnp
from jax.experimental import pallas as pl


def kernel(x):
    raise NotImplementedError("write your pallas kernel here")


# speedup vs baseline: 4.3057x; 4.3057x over previous
import functools

import jax
import jax.numpy as jnp
from jax import lax
from jax.experimental import pallas as pl
from jax.experimental.pallas import tpu as pltpu
from jax.experimental.pallas import tpu_sc as plsc

B = 1024
L = 200
NBINS = 512
HISTW = NBINS + 1
LANES = 16

NC, NS = 2, 16
NW = NC * NS
ROWS_PER_W = B // NW
ELEMS_PER_W = ROWS_PER_W * L
VREGS_PER_W = ELEMS_PER_W // LANES
OUT_PER_W = ROWS_PER_W * NBINS


def _hist_body(x_hbm, zeros_hbm, out_hbm, x_v, hist_v, out_v):
    wid = lax.axis_index("s") * NC + lax.axis_index("c")

    pltpu.sync_copy(x_hbm.at[pl.ds(wid * ELEMS_PER_W, ELEMS_PER_W)], x_v)
    pltpu.sync_copy(zeros_hbm, hist_v)

    ones = jnp.ones((LANES,), jnp.float32)
    lane_iota = lax.iota(jnp.int32, LANES)

    def scatter_step(j, _):
        vals = x_v[pl.ds(j * LANES, LANES)]
        pos = j * LANES + lane_iota
        row = pos // L
        binoff = jnp.where(vals >= 0, vals, NBINS)
        idx = row * HISTW + binoff
        plsc.addupdate_scatter(hist_v, [idx], ones)
        return _

    lax.fori_loop(0, VREGS_PER_W, scatter_step, None, unroll=4)

    def scale_row(r, _):
        tail = hist_v[pl.ds(r * HISTW + NBINS - (LANES - 1), LANES)]
        ign = tail[LANES - 1]
        den = jnp.full((LANES,), float(L), jnp.float32) - ign
        scale = jnp.where(den > 0, 1.0 / den, 0.0)

        def scale_step(k, _):
            h = hist_v[pl.ds(r * HISTW + k * LANES, LANES)]
            out_v[pl.ds(r * NBINS + k * LANES, LANES)] = h * scale
            return _

        lax.fori_loop(0, NBINS // LANES, scale_step, None, unroll=4)
        return _

    lax.fori_loop(0, ROWS_PER_W, scale_row, None)

    pltpu.sync_copy(out_v, out_hbm.at[pl.ds(wid * OUT_PER_W, OUT_PER_W)])


@jax.jit
def kernel(x):
    xf = x.reshape(-1).astype(jnp.int32)
    zeros = jnp.zeros((ROWS_PER_W * HISTW,), jnp.float32)
    mesh = plsc.VectorSubcoreMesh(core_axis_name="c", subcore_axis_name="s")
    out = pl.kernel(
        _hist_body,
        out_type=jax.ShapeDtypeStruct((B * NBINS,), jnp.float32),
        mesh=mesh,
        scratch_types=[
            pltpu.VMEM((ELEMS_PER_W,), jnp.int32),
            pltpu.VMEM((ROWS_PER_W * HISTW,), jnp.float32),
            pltpu.VMEM((OUT_PER_W,), jnp.float32),
        ],
        compiler_params=pltpu.CompilerParams(needs_layout_passes=False),
    )(xf, zeros)
    return out.reshape(B, NBINS)


# speedup vs baseline: 4.8439x; 1.1250x over previous
import jax
import jax.numpy as jnp
from jax import lax
from jax.experimental import pallas as pl
from jax.experimental.pallas import tpu as pltpu
from jax.experimental.pallas import tpu_sc as plsc

B = 1024
L = 200
NBINS = 512
HISTW = NBINS + 8
IGN = NBINS
LANES = 16
FULL_VREGS = L // LANES
TAIL_OFF = L - LANES

NC, NS = 2, 16
NW = NC * NS
ROWS_PER_W = B // NW


def _hist_body(x_hbm, zeros_hbm, out_hbm, x_v, hist_v, out_v, sem_x, sem_z):
    wid = lax.axis_index("s") * NC + lax.axis_index("c")
    row0 = wid * ROWS_PER_W

    cx = pltpu.async_copy(x_hbm.at[pl.ds(row0, ROWS_PER_W)], x_v, sem_x)
    cz = pltpu.async_copy(zeros_hbm, hist_v, sem_z)
    cx.wait()
    cz.wait()

    ones = jnp.ones((LANES,), jnp.float32)
    tail_mask = lax.iota(jnp.int32, LANES) >= (FULL_VREGS * LANES - TAIL_OFF)

    def do_row(r, _):
        rbase = r * HISTW
        for j in range(FULL_VREGS):
            vals = x_v[r, pl.ds(j * LANES, LANES)]
            binoff = jnp.where(vals >= 0, vals, NBINS)
            plsc.addupdate_scatter(hist_v, [rbase + binoff], ones)
        vals = x_v[r, pl.ds(TAIL_OFF, LANES)]
        binoff = jnp.where(vals >= 0, vals, NBINS)
        binoff = jnp.where(tail_mask, binoff, NBINS + 1)
        plsc.addupdate_scatter(hist_v, [rbase + binoff], ones)

        tail = hist_v[pl.ds(r * HISTW + IGN - 8, LANES)]
        ign = tail[8]
        den = jnp.full((LANES,), float(L), jnp.float32) - ign
        scale = jnp.where(den > 0, 1.0 / den, 0.0)

        def scale_step(k, _):
            h = hist_v[pl.ds(r * HISTW + k * LANES, LANES)]
            out_v[r, pl.ds(k * LANES, LANES)] = h * scale
            return _

        lax.fori_loop(0, NBINS // LANES, scale_step, None, unroll=8)
        return _

    lax.fori_loop(0, ROWS_PER_W, do_row, None)

    pltpu.sync_copy(out_v, out_hbm.at[pl.ds(row0, ROWS_PER_W)])


@jax.jit
def kernel(x):
    zeros = jnp.zeros((ROWS_PER_W * HISTW,), jnp.float32)
    mesh = plsc.VectorSubcoreMesh(core_axis_name="c", subcore_axis_name="s")
    out = pl.kernel(
        _hist_body,
        out_type=jax.ShapeDtypeStruct((B, NBINS), jnp.float32),
        mesh=mesh,
        scratch_types=[
            pltpu.VMEM((ROWS_PER_W, L), jnp.int32),
            pltpu.VMEM((ROWS_PER_W * HISTW,), jnp.float32),
            pltpu.VMEM((ROWS_PER_W, NBINS), jnp.float32),
            pltpu.SemaphoreType.DMA,
            pltpu.SemaphoreType.DMA,
        ],
        compiler_params=pltpu.CompilerParams(needs_layout_passes=False),
    )(x.astype(jnp.int32), zeros)
    return out


# speedup vs baseline: 5.2748x; 1.0890x over previous
import jax
import jax.numpy as jnp
from jax import lax
from jax.experimental import pallas as pl
from jax.experimental.pallas import tpu as pltpu
from jax.experimental.pallas import tpu_sc as plsc

B = 1024
L = 200
NBINS = 512
HISTW = NBINS + 8
IGN = NBINS
LANES = 16
FULL_VREGS = L // LANES
TAIL_OFF = L - LANES

NC, NS = 2, 16
NW = NC * NS
ROWS_PER_W = B // NW


def _hist_body(x_hbm, out_hbm, x_v, hist_v, out_v, sem_x):
    wid = lax.axis_index("s") * NC + lax.axis_index("c")
    row0 = wid * ROWS_PER_W

    cx = pltpu.async_copy(x_hbm.at[pl.ds(row0, ROWS_PER_W)], x_v, sem_x)

    zeros16 = jnp.zeros((LANES,), jnp.float32)

    def zero_step(i, _):
        hist_v[pl.ds(i * LANES, LANES)] = zeros16
        return _

    lax.fori_loop(0, ROWS_PER_W * HISTW // LANES, zero_step, None, unroll=8)
    cx.wait()

    ones = jnp.ones((LANES,), jnp.float32)
    tail_mask = lax.iota(jnp.int32, LANES) >= (FULL_VREGS * LANES - TAIL_OFF)

    def do_row(r, _):
        rbase = r * HISTW
        for j in range(FULL_VREGS):
            vals = x_v[r, pl.ds(j * LANES, LANES)]
            binoff = jnp.where(vals >= 0, vals, NBINS)
            plsc.addupdate_scatter(hist_v, [rbase + binoff], ones)
        vals = x_v[r, pl.ds(TAIL_OFF, LANES)]
        binoff = jnp.where(vals >= 0, vals, NBINS)
        binoff = jnp.where(tail_mask, binoff, NBINS + 1)
        plsc.addupdate_scatter(hist_v, [rbase + binoff], ones)

        tail = hist_v[pl.ds(r * HISTW + IGN - 8, LANES)]
        ign = tail[8]
        den = jnp.full((LANES,), float(L), jnp.float32) - ign
        scale = jnp.where(den > 0, 1.0 / den, 0.0)

        def scale_step(k, _):
            h = hist_v[pl.ds(r * HISTW + k * LANES, LANES)]
            out_v[r, pl.ds(k * LANES, LANES)] = h * scale
            return _

        lax.fori_loop(0, NBINS // LANES, scale_step, None, unroll=8)
        return _

    lax.fori_loop(0, ROWS_PER_W, do_row, None)

    pltpu.sync_copy(out_v, out_hbm.at[pl.ds(row0, ROWS_PER_W)])


@jax.jit
def kernel(x):
    mesh = plsc.VectorSubcoreMesh(core_axis_name="c", subcore_axis_name="s")
    out = pl.kernel(
        _hist_body,
        out_type=jax.ShapeDtypeStruct((B, NBINS), jnp.float32),
        mesh=mesh,
        scratch_types=[
            pltpu.VMEM((ROWS_PER_W, L), jnp.int32),
            pltpu.VMEM((ROWS_PER_W * HISTW,), jnp.float32),
            pltpu.VMEM((ROWS_PER_W, NBINS), jnp.float32),
            pltpu.SemaphoreType.DMA,
        ],
        compiler_params=pltpu.CompilerParams(needs_layout_passes=False),
    )(x.astype(jnp.int32))
    return out


# speedup vs baseline: 6.9810x; 1.3235x over previous
import jax
import jax.numpy as jnp
from jax import lax
from jax.experimental import pallas as pl
from jax.experimental.pallas import tpu as pltpu
from jax.experimental.pallas import tpu_sc as plsc

B = 1024
L = 200
NBINS = 512
LANES = 16
FULL_VREGS = L // LANES
TAIL_OFF = L - LANES

NC, NS = 2, 16
NW = NC * NS
ROWS_PER_W = B // NW


def _hist_body(x_hbm, out_hbm, x_v, out_v, sem_x):
    wid = lax.axis_index("s") * NC + lax.axis_index("c")
    row0 = wid * ROWS_PER_W

    cx = pltpu.async_copy(x_hbm.at[pl.ds(row0, ROWS_PER_W)], x_v, sem_x)

    zeros16 = jnp.zeros((LANES,), jnp.float32)

    def zero_step(i, _):
        out_v[i // (NBINS // LANES), pl.ds((i % (NBINS // LANES)) * LANES, LANES)] = zeros16
        return _

    lax.fori_loop(0, ROWS_PER_W * NBINS // LANES, zero_step, None, unroll=8)
    cx.wait()

    tail_keep = lax.iota(jnp.int32, LANES) >= (FULL_VREGS * LANES - TAIL_OFF)

    def do_row(r, _):
        vals = [x_v[r, pl.ds(j * LANES, LANES)] for j in range(FULL_VREGS)]
        vals.append(x_v[r, pl.ds(TAIL_OFF, LANES)])
        masks = [v >= 0 for v in vals[:FULL_VREGS]]
        masks.append((vals[FULL_VREGS] >= 0) & tail_keep)

        nvalid = plsc.all_reduce_population_count(masks[0])[0]
        for m in masks[1:]:
            nvalid = nvalid + plsc.all_reduce_population_count(m)[0]

        den = jnp.full((LANES,), nvalid, jnp.int32).astype(jnp.float32)
        scale = jnp.where(den > 0, 1.0 / den, 0.0)

        ridx = jnp.full((LANES,), r, jnp.int32)
        for v, m in zip(vals, masks):
            plsc.addupdate_scatter(out_v, [ridx, v], scale, mask=m)
        return _

    lax.fori_loop(0, ROWS_PER_W, do_row, None)

    pltpu.sync_copy(out_v, out_hbm.at[pl.ds(row0, ROWS_PER_W)])


@jax.jit
def kernel(x):
    mesh = plsc.VectorSubcoreMesh(core_axis_name="c", subcore_axis_name="s")
    out = pl.kernel(
        _hist_body,
        out_type=jax.ShapeDtypeStruct((B, NBINS), jnp.float32),
        mesh=mesh,
        scratch_types=[
            pltpu.VMEM((ROWS_PER_W, L), jnp.int32),
            pltpu.VMEM((ROWS_PER_W, NBINS), jnp.float32),
            pltpu.SemaphoreType.DMA,
        ],
        compiler_params=pltpu.CompilerParams(needs_layout_passes=False),
    )(x.astype(jnp.int32))
    return out


# speedup vs baseline: 6.9863x; 1.0008x over previous
import jax
import jax.numpy as jnp
from jax import lax
from jax.experimental import pallas as pl
from jax.experimental.pallas import tpu as pltpu
from jax.experimental.pallas import tpu_sc as plsc

B = 1024
L = 200
NBINS = 512
LANES = 16
FULL_VREGS = L // LANES
TAIL_OFF = L - LANES

NC, NS = 2, 16
NW = NC * NS
ROWS_PER_W = B // NW


def _hist_body(x_hbm, out_hbm, x_v, out_v, sem_x):
    wid = lax.axis_index("s") * NC + lax.axis_index("c")
    row0 = wid * ROWS_PER_W

    cx = pltpu.async_copy(x_hbm.at[pl.ds(row0, ROWS_PER_W)], x_v, sem_x)

    zeros16 = jnp.zeros((LANES,), jnp.float32)

    def zero_step(i, _):
        out_v[i // (NBINS // LANES), pl.ds((i % (NBINS // LANES)) * LANES, LANES)] = zeros16
        return _

    lax.fori_loop(0, ROWS_PER_W * NBINS // LANES, zero_step, None, unroll=8)
    cx.wait()

    tail_keep = lax.iota(jnp.int32, LANES) >= (FULL_VREGS * LANES - TAIL_OFF)

    def do_row(r, _):
        vals = [x_v[r, pl.ds(j * LANES, LANES)] for j in range(FULL_VREGS)]
        vals.append(x_v[r, pl.ds(TAIL_OFF, LANES)])
        masks = [v >= 0 for v in vals[:FULL_VREGS]]
        masks.append((vals[FULL_VREGS] >= 0) & tail_keep)

        nvalid = plsc.all_reduce_population_count(masks[0])[0]
        for m in masks[1:]:
            nvalid = nvalid + plsc.all_reduce_population_count(m)[0]

        den = jnp.full((LANES,), nvalid, jnp.int32).astype(jnp.float32)
        scale = jnp.where(den > 0, 1.0 / den, 0.0)

        ridx = jnp.full((LANES,), r, jnp.int32)
        for v, m in zip(vals, masks):
            plsc.addupdate_scatter(out_v, [ridx, v], scale, mask=m)
        return _

    lax.fori_loop(0, ROWS_PER_W, do_row, None)

    pltpu.sync_copy(out_v, out_hbm.at[pl.ds(row0, ROWS_PER_W)])


@jax.jit
def kernel(x):
    mesh = plsc.VectorSubcoreMesh(core_axis_name="c", subcore_axis_name="s")
    out = pl.kernel(
        _hist_body,
        out_type=jax.ShapeDtypeStruct((B, NBINS), jnp.float32),
        mesh=mesh,
        scratch_types=[
            pltpu.VMEM((ROWS_PER_W, L), jnp.int32),
            pltpu.VMEM((ROWS_PER_W, NBINS), jnp.float32),
            pltpu.SemaphoreType.DMA,
        ],
        compiler_params=pltpu.CompilerParams(needs_layout_passes=False, use_tc_tiling_on_sc=True),
    )(x.astype(jnp.int32))
    return out


# speedup vs baseline: 7.0494x; 1.0090x over previous
import jax
import jax.numpy as jnp
from jax import lax
from jax.experimental import pallas as pl
from jax.experimental.pallas import tpu as pltpu
from jax.experimental.pallas import tpu_sc as plsc

B = 1024
L = 200
NBINS = 512
LANES = 16
FULL_VREGS = L // LANES
TAIL_OFF = L - LANES

NC, NS = 2, 16
NW = NC * NS
ROWS_PER_W = B // NW


def _hist_body(x_hbm, out_hbm, x_v, out_v, sem_x):
    wid = lax.axis_index("s") * NC + lax.axis_index("c")
    row0 = wid * ROWS_PER_W

    cx = pltpu.async_copy(x_hbm.at[pl.ds(row0, ROWS_PER_W)], x_v, sem_x)

    zeros16 = jnp.zeros((LANES,), jnp.float32)

    def zero_step(i, _):
        out_v[i // (NBINS // LANES), pl.ds((i % (NBINS // LANES)) * LANES, LANES)] = zeros16
        return _

    lax.fori_loop(0, ROWS_PER_W * NBINS // LANES, zero_step, None, unroll=8)
    cx.wait()

    tail_keep = lax.iota(jnp.int32, LANES) >= (FULL_VREGS * LANES - TAIL_OFF)

    def do_row(r, _):
        vals = [x_v[r, pl.ds(j * LANES, LANES)] for j in range(FULL_VREGS)]
        vals.append(x_v[r, pl.ds(TAIL_OFF, LANES)])
        masks = [v >= 0 for v in vals[:FULL_VREGS]]
        masks.append((vals[FULL_VREGS] >= 0) & tail_keep)

        pcs = [plsc.all_reduce_population_count(m) for m in masks]
        while len(pcs) > 1:
            pcs = [a + b for a, b in zip(pcs[::2], pcs[1::2])] + (
                [pcs[-1]] if len(pcs) % 2 else [])
        den = pcs[0].astype(jnp.float32)
        scale = jnp.where(den > 0, 1.0 / den, 0.0)

        ridx = jnp.full((LANES,), r, jnp.int32)
        for v, m in zip(vals, masks):
            plsc.addupdate_scatter(out_v, [ridx, v], scale, mask=m)
        return _

    lax.fori_loop(0, ROWS_PER_W, do_row, None, unroll=2)

    pltpu.sync_copy(out_v, out_hbm.at[pl.ds(row0, ROWS_PER_W)])


@jax.jit
def kernel(x):
    mesh = plsc.VectorSubcoreMesh(core_axis_name="c", subcore_axis_name="s")
    out = pl.kernel(
        _hist_body,
        out_type=jax.ShapeDtypeStruct((B, NBINS), jnp.float32),
        mesh=mesh,
        scratch_types=[
            pltpu.VMEM((ROWS_PER_W, L), jnp.int32),
            pltpu.VMEM((ROWS_PER_W, NBINS), jnp.float32),
            pltpu.SemaphoreType.DMA,
        ],
        compiler_params=pltpu.CompilerParams(needs_layout_passes=False),
    )(x.astype(jnp.int32))
    return out


# speedup vs baseline: 7.0581x; 1.0012x over previous
import jax
import jax.numpy as jnp
from jax import lax
from jax.experimental import pallas as pl
from jax.experimental.pallas import tpu as pltpu
from jax.experimental.pallas import tpu_sc as plsc

B = 1024
L = 200
NBINS = 512
LANES = 16
FULL_VREGS = L // LANES
TAIL_OFF = L - LANES

NC, NS = 2, 16
NW = NC * NS
ROWS_PER_W = B // NW


def _hist_body(x_hbm, out_hbm, x_v, out_v, sem_x):
    wid = lax.axis_index("s") * NC + lax.axis_index("c")
    row0 = wid * ROWS_PER_W

    cx = pltpu.async_copy(x_hbm.at[pl.ds(row0, ROWS_PER_W)], x_v, sem_x)

    zeros16 = jnp.zeros((LANES,), jnp.float32)

    def zero_step(i, _):
        out_v[i // (NBINS // LANES), pl.ds((i % (NBINS // LANES)) * LANES, LANES)] = zeros16
        return _

    lax.fori_loop(0, ROWS_PER_W * NBINS // LANES, zero_step, None, unroll=8)
    cx.wait()

    tail_keep = lax.iota(jnp.int32, LANES) >= (FULL_VREGS * LANES - TAIL_OFF)

    def do_row(r, _):
        vals = [x_v[r, pl.ds(j * LANES, LANES)] for j in range(FULL_VREGS)]
        vals.append(x_v[r, pl.ds(TAIL_OFF, LANES)])
        masks = [v >= 0 for v in vals[:FULL_VREGS]]
        masks.append((vals[FULL_VREGS] >= 0) & tail_keep)

        pcs = [plsc.all_reduce_population_count(m) for m in masks]
        while len(pcs) > 1:
            pcs = [a + b for a, b in zip(pcs[::2], pcs[1::2])] + (
                [pcs[-1]] if len(pcs) % 2 else [])
        den = pcs[0].astype(jnp.float32)
        scale = jnp.where(den > 0, 1.0 / den, 0.0)

        ridx = jnp.full((LANES,), r, jnp.int32)
        for v, m in zip(vals, masks):
            plsc.addupdate_scatter(out_v, [ridx, v], scale, mask=m)
        return _

    lax.fori_loop(0, ROWS_PER_W, do_row, None, unroll=2)

    pltpu.sync_copy(out_v, out_hbm.at[pl.ds(row0, ROWS_PER_W)])


@jax.jit
def kernel(x):
    mesh = plsc.VectorSubcoreMesh(core_axis_name="c", subcore_axis_name="s")
    out = pl.kernel(
        _hist_body,
        out_type=jax.ShapeDtypeStruct((B, NBINS), jnp.float32),
        mesh=mesh,
        scratch_types=[
            pltpu.VMEM((ROWS_PER_W, L), jnp.int32),
            pltpu.VMEM((ROWS_PER_W, NBINS), jnp.float32),
            pltpu.SemaphoreType.DMA,
        ],
        compiler_params=pltpu.CompilerParams(needs_layout_passes=False, skip_device_barrier=True, disable_bounds_checks=True, disable_semaphore_checks=True),
    )(x.astype(jnp.int32))
    return out
